# Initial kernel scaffold; baseline (speedup 1.0000x reference)
import jax
import jax.numpy as jnp
from jax.experimental import pallas as pl


def kernel(state, edge_index, deterministic, Wc, bc, W1, b1, W2, b2, W3, b3):
    raise NotImplementedError("write your pallas kernel here")


# speedup vs baseline: 12.6542x; 12.6542x over previous
import functools

import jax
import jax.numpy as jnp
from jax import lax
from jax.experimental import pallas as pl
from jax.experimental.pallas import tpu as pltpu
from jax.experimental.pallas import tpu_sc as plsc

N = 10000
E = 320000
D = 128
H = 32
ACT = 8

NC = 2
NS = 16
NW = NC * NS
NPAD = 10240
RPT = NPAD // NS
EPT = E // NW
EB = 128
NB = 80

f32 = jnp.float32
i32 = jnp.int32


def _deg_body(dstp, cnt0, cnt1, idx_v, ones_v, zb_v, cnt_sh):
    cid = lax.axis_index("c")
    sid = lax.axis_index("s")
    wid = cid * NS + sid

    def zstep(j, _):
        zb_v[pl.ds(j * 16, 16)] = jnp.zeros((16,), f32)
        return 0

    lax.fori_loop(0, RPT // 16, zstep, 0)
    pltpu.sync_copy(zb_v, cnt_sh.at[pl.ds(sid * RPT, RPT)])

    def ostep(j, _):
        ones_v[pl.ds(j * 16, 16)] = jnp.ones((16,), f32)
        return 0

    lax.fori_loop(0, EB // 16, ostep, 0)
    pltpu.sync_copy(dstp.at[wid], idx_v)
    plsc.subcore_barrier()

    def step(j, _):
        pltpu.sync_copy(ones_v, cnt_sh.at[idx_v.at[j]], add=True)
        return 0

    lax.fori_loop(0, NB, step, 0)
    plsc.subcore_barrier()

    sl = pl.ds(sid * RPT, RPT)

    @pl.when(cid == 0)
    def _():
        pltpu.sync_copy(cnt_sh.at[sl], cnt0.at[sl])

    @pl.when(cid == 1)
    def _():
        pltpu.sync_copy(cnt_sh.at[sl], cnt1.at[sl])


def _count_deg(dstp):
    mesh = plsc.VectorSubcoreMesh(
        core_axis_name="c", subcore_axis_name="s", num_cores=NC, num_subcores=NS
    )
    return pl.kernel(
        _deg_body,
        out_type=(
            jax.ShapeDtypeStruct((NPAD,), f32),
            jax.ShapeDtypeStruct((NPAD,), f32),
        ),
        mesh=mesh,
        scratch_types=[
            pltpu.VMEM((NB, EB), i32),
            pltpu.VMEM((EB,), f32),
            pltpu.VMEM((RPT,), f32),
            pltpu.VMEM_SHARED((NPAD,), f32),
        ],
    )(dstp)


def _edge_body(g_hbm, srcp, dstp, out0, out1, srcv, dstv, rows_v, acc_sh, sem):
    cid = lax.axis_index("c")
    sid = lax.axis_index("s")
    wid = cid * NS + sid

    def zrow(r, _):
        for k in range(D // 16):
            rows_v[r, pl.ds(k * 16, 16)] = jnp.zeros((16,), f32)
        return 0

    lax.fori_loop(0, EB, zrow, 0)
    for k in range(RPT // EB):
        pltpu.sync_copy(rows_v, acc_sh.at[pl.ds(sid * RPT + k * EB, EB)])

    pltpu.sync_copy(srcp.at[wid], srcv)
    pltpu.sync_copy(dstp.at[wid], dstv)
    plsc.subcore_barrier()

    def step(j, _):
        pltpu.async_copy(g_hbm.at[srcv.at[j]], rows_v, sem).wait()
        pltpu.sync_copy(rows_v, acc_sh.at[dstv.at[j]], add=True)
        return 0

    lax.fori_loop(0, NB, step, 0)
    plsc.subcore_barrier()

    sl = pl.ds(sid * RPT, RPT)

    @pl.when(cid == 0)
    def _():
        pltpu.sync_copy(acc_sh.at[sl], out0.at[sl])

    @pl.when(cid == 1)
    def _():
        pltpu.sync_copy(acc_sh.at[sl], out1.at[sl])


def _edge_pass(g, srcp, dstp):
    mesh = plsc.VectorSubcoreMesh(
        core_axis_name="c", subcore_axis_name="s", num_cores=NC, num_subcores=NS
    )
    return pl.kernel(
        _edge_body,
        out_type=(
            jax.ShapeDtypeStruct((NPAD, D), f32),
            jax.ShapeDtypeStruct((NPAD, D), f32),
        ),
        mesh=mesh,
        scratch_types=[
            pltpu.VMEM((NB, EB), i32),
            pltpu.VMEM((NB, EB), i32),
            pltpu.VMEM((EB, D), f32),
            pltpu.VMEM_SHARED((NPAD, D), f32),
            pltpu.SemaphoreType.DMA,
        ],
    )(g, srcp, dstp)


_BR = 1280
_GRID = NPAD // _BR


def _b_body(x_ref, wc_ref, c0_ref, c1_ref, g_ref, d_ref):
    h = jnp.dot(x_ref[...], wc_ref[...], preferred_element_type=f32)
    deg = 1.0 + c0_ref[...] + c1_ref[...]
    dv = lax.rsqrt(deg)
    d_ref[...] = dv
    g_ref[...] = h * dv


def _premix(state_p, Wc, c0, c1):
    return pl.pallas_call(
        _b_body,
        grid=(_GRID,),
        in_specs=[
            pl.BlockSpec((_BR, D), lambda i: (i, 0)),
            pl.BlockSpec((D, D), lambda i: (0, 0)),
            pl.BlockSpec((_BR, 1), lambda i: (i, 0)),
            pl.BlockSpec((_BR, 1), lambda i: (i, 0)),
        ],
        out_specs=[
            pl.BlockSpec((_BR, D), lambda i: (i, 0)),
            pl.BlockSpec((_BR, 1), lambda i: (i, 0)),
        ],
        out_shape=[
            jax.ShapeDtypeStruct((NPAD, D), f32),
            jax.ShapeDtypeStruct((NPAD, 1), f32),
        ],
    )(state_p, Wc, c0, c1)


def _d_body(a0, a1, g, x0, d, bc, w1, b1, w2, b2, w3, b3, conc_ref, sum_ref, acc):
    i = pl.program_id(0)
    a = a0[...] + a1[...] + g[...]
    out = jnp.maximum(d[...] * a + bc[...], 0.0)
    x = out + x0[...]
    y = jnp.dot(x, w1[...], preferred_element_type=f32) + b1[...]
    y = jnp.where(y >= 0, y, 0.01 * y)
    y = jnp.dot(y, w2[...], preferred_element_type=f32) + b2[...]
    y = jnp.where(y >= 0, y, 0.01 * y)
    z = jnp.dot(y, w3[...], preferred_element_type=f32) + b3[...]
    sp = jnp.maximum(z, 0.0) + jnp.log(1.0 + jnp.exp(-jnp.abs(z)))
    row = i * _BR + lax.broadcasted_iota(i32, (_BR, 1), 0)
    sp = jnp.where(row < N, sp, 0.0)
    conc_ref[...] = sp
    prev = jnp.where(i == 0, 0.0, acc[0, 0])
    tot = prev + jnp.sum(sp)
    acc[0, 0] = tot
    sum_ref[...] = jnp.broadcast_to(tot, (1, 1))


def _head(a0, a1, g, state_p, d, bc2, W1, b12, W2, b22, W3, b32):
    return pl.pallas_call(
        _d_body,
        grid=(_GRID,),
        in_specs=[
            pl.BlockSpec((_BR, D), lambda i: (i, 0)),
            pl.BlockSpec((_BR, D), lambda i: (i, 0)),
            pl.BlockSpec((_BR, D), lambda i: (i, 0)),
            pl.BlockSpec((_BR, D), lambda i: (i, 0)),
            pl.BlockSpec((_BR, 1), lambda i: (i, 0)),
            pl.BlockSpec((1, D), lambda i: (0, 0)),
            pl.BlockSpec((D, H), lambda i: (0, 0)),
            pl.BlockSpec((1, H), lambda i: (0, 0)),
            pl.BlockSpec((H, H), lambda i: (0, 0)),
            pl.BlockSpec((1, H), lambda i: (0, 0)),
            pl.BlockSpec((H, 1), lambda i: (0, 0)),
            pl.BlockSpec((1, 1), lambda i: (0, 0)),
        ],
        out_specs=[
            pl.BlockSpec((_BR, 1), lambda i: (i, 0)),
            pl.BlockSpec((1, 1), lambda i: (0, 0)),
        ],
        out_shape=[
            jax.ShapeDtypeStruct((NPAD, 1), f32),
            jax.ShapeDtypeStruct((1, 1), f32),
        ],
        scratch_shapes=[pltpu.SMEM((1, 1), f32)],
    )(a0, a1, g, state_p, d, bc2, W1, b12, W2, b22, W3, b32)


def _e_body(conc_ref, s_ref, out_ref):
    out_ref[...] = conc_ref[...] / (s_ref[...] + 1e-20)


def _normalize(conc, ssum):
    return pl.pallas_call(
        _e_body,
        grid=(_GRID,),
        in_specs=[
            pl.BlockSpec((_BR, 1), lambda i: (i, 0)),
            pl.BlockSpec((1, 1), lambda i: (0, 0)),
        ],
        out_specs=pl.BlockSpec((_BR, 1), lambda i: (i, 0)),
        out_shape=jax.ShapeDtypeStruct((NPAD, 1), f32),
    )(conc, ssum)


def kernel(state, edge_index, deterministic, Wc, bc, W1, b1, W2, b2, W3, b3):
    src = edge_index[0].reshape(NW, EPT)
    dst = edge_index[1].reshape(NW, EPT)
    padw = NB * EB - EPT
    srcp = jnp.concatenate(
        [src, jnp.zeros((NW, padw), i32)], axis=1
    ).reshape(NW, NB, EB)
    dstp = jnp.concatenate(
        [dst, jnp.full((NW, padw), N, i32)], axis=1
    ).reshape(NW, NB, EB)
    state_p = jnp.concatenate([state, jnp.zeros((NPAD - N, D), f32)], axis=0)

    c0, c1 = _count_deg(dstp)
    g, d = _premix(state_p, Wc, c0.reshape(NPAD, 1), c1.reshape(NPAD, 1))
    a0, a1 = _edge_pass(g, srcp, dstp)
    conc, ssum = _head(
        a0, a1, g, state_p, d,
        bc.reshape(1, D),
        W1, b1.reshape(1, H),
        W2, b2.reshape(1, H),
        W3, b3.reshape(1, 1),
    )
    act = _normalize(conc, ssum)
    return act[:N, 0].reshape(N // ACT, ACT)


# speedup vs baseline: 14.2640x; 1.1272x over previous
import functools

import jax
import jax.numpy as jnp
from jax import lax
from jax.experimental import pallas as pl
from jax.experimental.pallas import tpu as pltpu
from jax.experimental.pallas import tpu_sc as plsc

N = 10000
E = 320000
D = 128
H = 32
ACT = 8

NC = 2
NS = 16
NW = NC * NS
NPAD = 10240
RPT = NPAD // NS
EPT = E // NW
EB = 128
NB = 80

f32 = jnp.float32
i32 = jnp.int32


def _deg_body(dstp, cnt0, cnt1, idx_v, ones_v, zb_v, cnt_sh):
    cid = lax.axis_index("c")
    sid = lax.axis_index("s")
    wid = cid * NS + sid

    def zstep(j, _):
        zb_v[pl.ds(j * 16, 16)] = jnp.zeros((16,), f32)
        return 0

    lax.fori_loop(0, RPT // 16, zstep, 0)
    pltpu.sync_copy(zb_v, cnt_sh.at[pl.ds(sid * RPT, RPT)])

    def ostep(j, _):
        ones_v[pl.ds(j * 16, 16)] = jnp.ones((16,), f32)
        return 0

    lax.fori_loop(0, EB // 16, ostep, 0)
    pltpu.sync_copy(dstp.at[wid], idx_v)
    plsc.subcore_barrier()

    def step(j, _):
        pltpu.sync_copy(ones_v, cnt_sh.at[idx_v.at[j]], add=True)
        return 0

    lax.fori_loop(0, NB, step, 0)
    plsc.subcore_barrier()

    sl = pl.ds(sid * RPT, RPT)

    @pl.when(cid == 0)
    def _():
        pltpu.sync_copy(cnt_sh.at[sl], cnt0.at[sl])

    @pl.when(cid == 1)
    def _():
        pltpu.sync_copy(cnt_sh.at[sl], cnt1.at[sl])


def _count_deg(dstp):
    mesh = plsc.VectorSubcoreMesh(
        core_axis_name="c", subcore_axis_name="s", num_cores=NC, num_subcores=NS
    )
    return pl.kernel(
        _deg_body,
        out_type=(
            jax.ShapeDtypeStruct((NPAD,), f32),
            jax.ShapeDtypeStruct((NPAD,), f32),
        ),
        mesh=mesh,
        scratch_types=[
            pltpu.VMEM((NB, EB), i32),
            pltpu.VMEM((EB,), f32),
            pltpu.VMEM((RPT,), f32),
            pltpu.VMEM_SHARED((NPAD,), f32),
        ],
    )(dstp)


def _edge_body(
    g_hbm, srcp, dstp, out0, out1, srcv, dstv, rows0, rows1, acc_sh, sem0, sem1
):
    cid = lax.axis_index("c")
    sid = lax.axis_index("s")
    wid = cid * NS + sid

    def zrow(r, _):
        for k in range(D // 16):
            rows0[r, pl.ds(k * 16, 16)] = jnp.zeros((16,), f32)
        return 0

    lax.fori_loop(0, EB, zrow, 0)
    for k in range(RPT // EB):
        pltpu.sync_copy(rows0, acc_sh.at[pl.ds(sid * RPT + k * EB, EB)])

    plsc.subcore_barrier()

    HB = NB // 2
    for c in range(2):
        pltpu.sync_copy(srcp.at[wid, pl.ds(c * HB, HB)], srcv)
        pltpu.sync_copy(dstp.at[wid, pl.ds(c * HB, HB)], dstv)
        pltpu.async_copy(g_hbm.at[srcv.at[0]], rows0, sem0)
        pltpu.async_copy(g_hbm.at[srcv.at[1]], rows1, sem1)

        def step(jj, _):
            for b, rows, sem in ((0, rows0, sem0), (1, rows1, sem1)):
                batch = jj * 2 + b
                pltpu.make_async_copy(g_hbm.at[srcv.at[batch]], rows, sem).wait()
                pltpu.sync_copy(rows, acc_sh.at[dstv.at[batch]], add=True)

                @pl.when(batch + 2 < HB)
                def _():
                    pltpu.async_copy(g_hbm.at[srcv.at[batch + 2]], rows, sem)

            return 0

        lax.fori_loop(0, HB // 2, step, 0)
    plsc.subcore_barrier()

    sl = pl.ds(sid * RPT, RPT)

    @pl.when(cid == 0)
    def _():
        pltpu.sync_copy(acc_sh.at[sl], out0.at[sl])

    @pl.when(cid == 1)
    def _():
        pltpu.sync_copy(acc_sh.at[sl], out1.at[sl])


def _edge_pass(g, srcp, dstp):
    mesh = plsc.VectorSubcoreMesh(
        core_axis_name="c", subcore_axis_name="s", num_cores=NC, num_subcores=NS
    )
    return pl.kernel(
        _edge_body,
        out_type=(
            jax.ShapeDtypeStruct((NPAD, D), f32),
            jax.ShapeDtypeStruct((NPAD, D), f32),
        ),
        mesh=mesh,
        scratch_types=[
            pltpu.VMEM((NB // 2, EB), i32),
            pltpu.VMEM((NB // 2, EB), i32),
            pltpu.VMEM((EB, D), f32),
            pltpu.VMEM((EB, D), f32),
            pltpu.VMEM_SHARED((NPAD, D), f32),
            pltpu.SemaphoreType.DMA,
            pltpu.SemaphoreType.DMA,
        ],
    )(g, srcp, dstp)


_BR = 1280
_GRID = NPAD // _BR


def _b_body(x_ref, wc_ref, c0_ref, c1_ref, g_ref, d_ref):
    h = jnp.dot(x_ref[...], wc_ref[...], preferred_element_type=f32)
    deg = 1.0 + c0_ref[...] + c1_ref[...]
    dv = lax.rsqrt(deg)
    d_ref[...] = dv
    g_ref[...] = h * dv


def _premix(state_p, Wc, c0, c1):
    return pl.pallas_call(
        _b_body,
        grid=(_GRID,),
        in_specs=[
            pl.BlockSpec((_BR, D), lambda i: (i, 0)),
            pl.BlockSpec((D, D), lambda i: (0, 0)),
            pl.BlockSpec((_BR, 1), lambda i: (i, 0)),
            pl.BlockSpec((_BR, 1), lambda i: (i, 0)),
        ],
        out_specs=[
            pl.BlockSpec((_BR, D), lambda i: (i, 0)),
            pl.BlockSpec((_BR, 1), lambda i: (i, 0)),
        ],
        out_shape=[
            jax.ShapeDtypeStruct((NPAD, D), f32),
            jax.ShapeDtypeStruct((NPAD, 1), f32),
        ],
    )(state_p, Wc, c0, c1)


def _d_body(a0, a1, g, x0, d, bc, w1, b1, w2, b2, w3, b3, conc_ref, sum_ref, acc):
    i = pl.program_id(0)
    a = a0[...] + a1[...] + g[...]
    out = jnp.maximum(d[...] * a + bc[...], 0.0)
    x = out + x0[...]
    y = jnp.dot(x, w1[...], preferred_element_type=f32) + b1[...]
    y = jnp.where(y >= 0, y, 0.01 * y)
    y = jnp.dot(y, w2[...], preferred_element_type=f32) + b2[...]
    y = jnp.where(y >= 0, y, 0.01 * y)
    z = jnp.dot(y, w3[...], preferred_element_type=f32) + b3[...]
    sp = jnp.maximum(z, 0.0) + jnp.log(1.0 + jnp.exp(-jnp.abs(z)))
    row = i * _BR + lax.broadcasted_iota(i32, (_BR, 1), 0)
    sp = jnp.where(row < N, sp, 0.0)
    conc_ref[...] = sp
    prev = jnp.where(i == 0, 0.0, acc[0, 0])
    tot = prev + jnp.sum(sp)
    acc[0, 0] = tot
    sum_ref[...] = jnp.broadcast_to(tot, (1, 1))


def _head(a0, a1, g, state_p, d, bc2, W1, b12, W2, b22, W3, b32):
    return pl.pallas_call(
        _d_body,
        grid=(_GRID,),
        in_specs=[
            pl.BlockSpec((_BR, D), lambda i: (i, 0)),
            pl.BlockSpec((_BR, D), lambda i: (i, 0)),
            pl.BlockSpec((_BR, D), lambda i: (i, 0)),
            pl.BlockSpec((_BR, D), lambda i: (i, 0)),
            pl.BlockSpec((_BR, 1), lambda i: (i, 0)),
            pl.BlockSpec((1, D), lambda i: (0, 0)),
            pl.BlockSpec((D, H), lambda i: (0, 0)),
            pl.BlockSpec((1, H), lambda i: (0, 0)),
            pl.BlockSpec((H, H), lambda i: (0, 0)),
            pl.BlockSpec((1, H), lambda i: (0, 0)),
            pl.BlockSpec((H, 1), lambda i: (0, 0)),
            pl.BlockSpec((1, 1), lambda i: (0, 0)),
        ],
        out_specs=[
            pl.BlockSpec((_BR, 1), lambda i: (i, 0)),
            pl.BlockSpec((1, 1), lambda i: (0, 0)),
        ],
        out_shape=[
            jax.ShapeDtypeStruct((NPAD, 1), f32),
            jax.ShapeDtypeStruct((1, 1), f32),
        ],
        scratch_shapes=[pltpu.SMEM((1, 1), f32)],
    )(a0, a1, g, state_p, d, bc2, W1, b12, W2, b22, W3, b32)


def _e_body(conc_ref, s_ref, out_ref):
    out_ref[...] = conc_ref[...] / (s_ref[...] + 1e-20)


def _normalize(conc, ssum):
    return pl.pallas_call(
        _e_body,
        grid=(_GRID,),
        in_specs=[
            pl.BlockSpec((_BR, 1), lambda i: (i, 0)),
            pl.BlockSpec((1, 1), lambda i: (0, 0)),
        ],
        out_specs=pl.BlockSpec((_BR, 1), lambda i: (i, 0)),
        out_shape=jax.ShapeDtypeStruct((NPAD, 1), f32),
    )(conc, ssum)


def kernel(state, edge_index, deterministic, Wc, bc, W1, b1, W2, b2, W3, b3):
    src = edge_index[0].reshape(NW, EPT)
    dst = edge_index[1].reshape(NW, EPT)
    padw = NB * EB - EPT
    srcp = jnp.concatenate(
        [src, jnp.zeros((NW, padw), i32)], axis=1
    ).reshape(NW, NB, EB)
    dstp = jnp.concatenate(
        [dst, jnp.full((NW, padw), N, i32)], axis=1
    ).reshape(NW, NB, EB)
    state_p = jnp.concatenate([state, jnp.zeros((NPAD - N, D), f32)], axis=0)

    c0, c1 = _count_deg(dstp)
    g, d = _premix(state_p, Wc, c0.reshape(NPAD, 1), c1.reshape(NPAD, 1))
    a0, a1 = _edge_pass(g, srcp, dstp)
    conc, ssum = _head(
        a0, a1, g, state_p, d,
        bc.reshape(1, D),
        W1, b1.reshape(1, H),
        W2, b2.reshape(1, H),
        W3, b3.reshape(1, 1),
    )
    act = _normalize(conc, ssum)
    return act[:N, 0].reshape(N // ACT, ACT)


# speedup vs baseline: 14.3773x; 1.0079x over previous
import jax
import jax.numpy as jnp
from jax import lax
from jax.experimental import pallas as pl
from jax.experimental.pallas import tpu as pltpu
from jax.experimental.pallas import tpu_sc as plsc

N = 10000
E = 320000
D = 128
H = 32
ACT = 8

NC = 2
NS = 16
NW = NC * NS
NPAD = 10240
RPT = NPAD // NS
EPT = E // NW
EB = 128
NB = 80

f32 = jnp.float32
i32 = jnp.int32


def _deg_body(dstp, cnt0, cnt1, idx_v, ones_v, zb_v, cnt_sh):
    cid = lax.axis_index("c")
    sid = lax.axis_index("s")
    wid = cid * NS + sid

    def zstep(j, _):
        zb_v[pl.ds(j * 16, 16)] = jnp.zeros((16,), f32)
        return 0

    lax.fori_loop(0, RPT // 16, zstep, 0)
    pltpu.sync_copy(zb_v, cnt_sh.at[pl.ds(sid * RPT, RPT)])

    def ostep(j, _):
        ones_v[pl.ds(j * 16, 16)] = jnp.ones((16,), f32)
        return 0

    lax.fori_loop(0, EB // 16, ostep, 0)
    pltpu.sync_copy(dstp.at[wid], idx_v)
    plsc.subcore_barrier()

    def step(j, _):
        pltpu.sync_copy(ones_v, cnt_sh.at[idx_v.at[j]], add=True)
        return 0

    lax.fori_loop(0, NB, step, 0)
    plsc.subcore_barrier()

    sl = pl.ds(sid * RPT, RPT)

    @pl.when(cid == 0)
    def _():
        pltpu.sync_copy(cnt_sh.at[sl], cnt0.at[sl])

    @pl.when(cid == 1)
    def _():
        pltpu.sync_copy(cnt_sh.at[sl], cnt1.at[sl])


def _count_deg(dstp):
    mesh = plsc.VectorSubcoreMesh(
        core_axis_name="c", subcore_axis_name="s", num_cores=NC, num_subcores=NS
    )
    return pl.kernel(
        _deg_body,
        out_type=(
            jax.ShapeDtypeStruct((NPAD,), f32),
            jax.ShapeDtypeStruct((NPAD,), f32),
        ),
        mesh=mesh,
        scratch_types=[
            pltpu.VMEM((NB, EB), i32),
            pltpu.VMEM((EB,), f32),
            pltpu.VMEM((RPT,), f32),
            pltpu.VMEM_SHARED((NPAD,), f32),
        ],
    )(dstp)


def _edge_body(
    g_hbm, srcp, dstp, out0, out1, srcv, dstv, rows0, rows1, acc_sh, sem0, sem1
):
    cid = lax.axis_index("c")
    sid = lax.axis_index("s")
    wid = cid * NS + sid

    def zrow(r, _):
        for k in range(D // 16):
            rows0[r, pl.ds(k * 16, 16)] = jnp.zeros((16,), f32)
        return 0

    lax.fori_loop(0, EB, zrow, 0)
    for k in range(RPT // EB):
        pltpu.sync_copy(rows0, acc_sh.at[pl.ds(sid * RPT + k * EB, EB)])

    plsc.subcore_barrier()

    HB = NB // 2
    for c in range(2):
        pltpu.sync_copy(srcp.at[wid, pl.ds(c * HB, HB)], srcv)
        pltpu.sync_copy(dstp.at[wid, pl.ds(c * HB, HB)], dstv)
        pltpu.async_copy(g_hbm.at[srcv.at[0]], rows0, sem0)
        pltpu.async_copy(g_hbm.at[srcv.at[1]], rows1, sem1)

        def step(jj, _):
            for b, rows, sem in ((0, rows0, sem0), (1, rows1, sem1)):
                batch = jj * 2 + b
                pltpu.make_async_copy(g_hbm.at[srcv.at[batch]], rows, sem).wait()
                pltpu.sync_copy(rows, acc_sh.at[dstv.at[batch]], add=True)

                @pl.when(batch + 2 < HB)
                def _():
                    pltpu.async_copy(g_hbm.at[srcv.at[batch + 2]], rows, sem)

            return 0

        lax.fori_loop(0, HB // 2, step, 0)
    plsc.subcore_barrier()

    sl = pl.ds(sid * RPT, RPT)

    @pl.when(cid == 0)
    def _():
        pltpu.sync_copy(acc_sh.at[sl], out0.at[sl])

    @pl.when(cid == 1)
    def _():
        pltpu.sync_copy(acc_sh.at[sl], out1.at[sl])


def _edge_pass(g, srcp, dstp):
    mesh = plsc.VectorSubcoreMesh(
        core_axis_name="c", subcore_axis_name="s", num_cores=NC, num_subcores=NS
    )
    return pl.kernel(
        _edge_body,
        out_type=(
            jax.ShapeDtypeStruct((NPAD, D), f32),
            jax.ShapeDtypeStruct((NPAD, D), f32),
        ),
        mesh=mesh,
        scratch_types=[
            pltpu.VMEM((NB // 2, EB), i32),
            pltpu.VMEM((NB // 2, EB), i32),
            pltpu.VMEM((EB, D), f32),
            pltpu.VMEM((EB, D), f32),
            pltpu.VMEM_SHARED((NPAD, D), f32),
            pltpu.SemaphoreType.DMA,
            pltpu.SemaphoreType.DMA,
        ],
    )(g, srcp, dstp)


_BR = 2000
_GRID = N // _BR


def _b_body(x_ref, wc_ref, c0_ref, c1_ref, g_ref, d_ref):
    h = jnp.dot(x_ref[...], wc_ref[...], preferred_element_type=f32)
    deg = 1.0 + c0_ref[...] + c1_ref[...]
    dv = lax.rsqrt(deg)
    d_ref[...] = dv
    g_ref[...] = h * dv


def _premix(state, Wc, c0, c1):
    return pl.pallas_call(
        _b_body,
        grid=(_GRID,),
        in_specs=[
            pl.BlockSpec((_BR, D), lambda i: (i, 0)),
            pl.BlockSpec((D, D), lambda i: (0, 0)),
            pl.BlockSpec((_BR, 1), lambda i: (i, 0)),
            pl.BlockSpec((_BR, 1), lambda i: (i, 0)),
        ],
        out_specs=[
            pl.BlockSpec((_BR, D), lambda i: (i, 0)),
            pl.BlockSpec((_BR, 1), lambda i: (i, 0)),
        ],
        out_shape=[
            jax.ShapeDtypeStruct((N, D), f32),
            jax.ShapeDtypeStruct((N, 1), f32),
        ],
    )(state, Wc, c0, c1)


def _d_body(a0, a1, g, x0, d, bc, w1, b1, w2, b2, w3, b3, conc_ref, sum_ref, acc):
    i = pl.program_id(0)
    a = a0[...] + a1[...] + g[...]
    out = jnp.maximum(d[...] * a + bc[...], 0.0)
    x = out + x0[...]
    y = jnp.dot(x, w1[...], preferred_element_type=f32) + b1[...]
    y = jnp.where(y >= 0, y, 0.01 * y)
    y = jnp.dot(y, w2[...], preferred_element_type=f32) + b2[...]
    y = jnp.where(y >= 0, y, 0.01 * y)
    z = jnp.dot(y, w3[...], preferred_element_type=f32) + b3[...]
    sp = jnp.maximum(z, 0.0) + jnp.log(1.0 + jnp.exp(-jnp.abs(z)))
    conc_ref[...] = sp
    prev = jnp.where(i == 0, 0.0, acc[0, 0])
    tot = prev + jnp.sum(sp)
    acc[0, 0] = tot
    sum_ref[...] = jnp.broadcast_to(tot, (1, 1))


def _head(a0, a1, g, state, d, bc2, W1, b12, W2, b22, W3, b32):
    return pl.pallas_call(
        _d_body,
        grid=(_GRID,),
        in_specs=[
            pl.BlockSpec((_BR, D), lambda i: (i, 0)),
            pl.BlockSpec((_BR, D), lambda i: (i, 0)),
            pl.BlockSpec((_BR, D), lambda i: (i, 0)),
            pl.BlockSpec((_BR, D), lambda i: (i, 0)),
            pl.BlockSpec((_BR, 1), lambda i: (i, 0)),
            pl.BlockSpec((1, D), lambda i: (0, 0)),
            pl.BlockSpec((D, H), lambda i: (0, 0)),
            pl.BlockSpec((1, H), lambda i: (0, 0)),
            pl.BlockSpec((H, H), lambda i: (0, 0)),
            pl.BlockSpec((1, H), lambda i: (0, 0)),
            pl.BlockSpec((H, 1), lambda i: (0, 0)),
            pl.BlockSpec((1, 1), lambda i: (0, 0)),
        ],
        out_specs=[
            pl.BlockSpec((_BR, 1), lambda i: (i, 0)),
            pl.BlockSpec((1, 1), lambda i: (0, 0)),
        ],
        out_shape=[
            jax.ShapeDtypeStruct((N, 1), f32),
            jax.ShapeDtypeStruct((1, 1), f32),
        ],
        scratch_shapes=[pltpu.SMEM((1, 1), f32)],
    )(a0, a1, g, state, d, bc2, W1, b12, W2, b22, W3, b32)


def _e_body(conc_ref, s_ref, out_ref):
    out_ref[...] = conc_ref[...] / (s_ref[...] + 1e-20)


def _normalize(conc, ssum):
    return pl.pallas_call(
        _e_body,
        grid=(_GRID,),
        in_specs=[
            pl.BlockSpec((_BR, 1), lambda i: (i, 0)),
            pl.BlockSpec((1, 1), lambda i: (0, 0)),
        ],
        out_specs=pl.BlockSpec((_BR, 1), lambda i: (i, 0)),
        out_shape=jax.ShapeDtypeStruct((N, 1), f32),
    )(conc, ssum)


def kernel(state, edge_index, deterministic, Wc, bc, W1, b1, W2, b2, W3, b3):
    src = edge_index[0].reshape(NW, EPT)
    dst = edge_index[1].reshape(NW, EPT)
    padw = NB * EB - EPT
    srcp = jnp.concatenate(
        [src, jnp.zeros((NW, padw), i32)], axis=1
    ).reshape(NW, NB, EB)
    dstp = jnp.concatenate(
        [dst, jnp.full((NW, padw), N, i32)], axis=1
    ).reshape(NW, NB, EB)

    c0, c1 = _count_deg(dstp)
    g, d = _premix(state, Wc, c0.reshape(NPAD, 1), c1.reshape(NPAD, 1))
    a0, a1 = _edge_pass(g, srcp, dstp)
    conc, ssum = _head(
        a0, a1, g, state, d,
        bc.reshape(1, D),
        W1, b1.reshape(1, H),
        W2, b2.reshape(1, H),
        W3, b3.reshape(1, 1),
    )
    act = _normalize(conc, ssum)
    return act[:, 0].reshape(N // ACT, ACT)


# speedup vs baseline: 14.4799x; 1.0071x over previous
import jax
import jax.numpy as jnp
from jax import lax
from jax.experimental import pallas as pl
from jax.experimental.pallas import tpu as pltpu
from jax.experimental.pallas import tpu_sc as plsc

N = 10000
E = 320000
D = 128
H = 32
ACT = 8

NC = 2
NS = 16
NW = NC * NS
NPAD = 10240
RPT = NPAD // NS
EPT = E // NW
EB = 128
NB = 80
EB2 = 64
NB2 = 160
QB2 = NB2 // 4

f32 = jnp.float32
i32 = jnp.int32


def _deg_body(dstp, cnt0, cnt1, idx_v, ones_v, zb_v, cnt_sh):
    cid = lax.axis_index("c")
    sid = lax.axis_index("s")
    wid = cid * NS + sid

    def zstep(j, _):
        zb_v[pl.ds(j * 16, 16)] = jnp.zeros((16,), f32)
        return 0

    lax.fori_loop(0, RPT // 16, zstep, 0)
    pltpu.sync_copy(zb_v, cnt_sh.at[pl.ds(sid * RPT, RPT)])

    def ostep(j, _):
        ones_v[pl.ds(j * 16, 16)] = jnp.ones((16,), f32)
        return 0

    lax.fori_loop(0, EB // 16, ostep, 0)
    pltpu.sync_copy(dstp.at[wid], idx_v)
    plsc.subcore_barrier()

    def step(j, _):
        pltpu.sync_copy(ones_v, cnt_sh.at[idx_v.at[j]], add=True)
        return 0

    lax.fori_loop(0, NB, step, 0)
    plsc.subcore_barrier()

    sl = pl.ds(sid * RPT, RPT)

    @pl.when(cid == 0)
    def _():
        pltpu.sync_copy(cnt_sh.at[sl], cnt0.at[sl])

    @pl.when(cid == 1)
    def _():
        pltpu.sync_copy(cnt_sh.at[sl], cnt1.at[sl])


def _count_deg(dstp):
    mesh = plsc.VectorSubcoreMesh(
        core_axis_name="c", subcore_axis_name="s", num_cores=NC, num_subcores=NS
    )
    return pl.kernel(
        _deg_body,
        out_type=(
            jax.ShapeDtypeStruct((NPAD,), f32),
            jax.ShapeDtypeStruct((NPAD,), f32),
        ),
        mesh=mesh,
        scratch_types=[
            pltpu.VMEM((NB, EB), i32),
            pltpu.VMEM((EB,), f32),
            pltpu.VMEM((RPT,), f32),
            pltpu.VMEM_SHARED((NPAD,), f32),
        ],
    )(dstp)


def _edge_body(
    g_hbm, srcp, dstp, out0, out1,
    srcv, dstv, rows0, rows1, rows2, rows3, acc_sh, sem0, sem1, sem2, sem3
):
    cid = lax.axis_index("c")
    sid = lax.axis_index("s")
    wid = cid * NS + sid
    ring = ((rows0, sem0), (rows1, sem1), (rows2, sem2), (rows3, sem3))

    def zrow(r, _):
        for k in range(D // 16):
            rows0[r, pl.ds(k * 16, 16)] = jnp.zeros((16,), f32)
            rows1[r, pl.ds(k * 16, 16)] = jnp.zeros((16,), f32)
        return 0

    lax.fori_loop(0, EB2, zrow, 0)
    for k in range(RPT // (2 * EB2)):
        pltpu.sync_copy(rows0, acc_sh.at[pl.ds(sid * RPT + 2 * k * EB2, EB2)])
        pltpu.sync_copy(
            rows1, acc_sh.at[pl.ds(sid * RPT + (2 * k + 1) * EB2, EB2)]
        )

    plsc.subcore_barrier()

    for ph in range(4):
        pltpu.sync_copy(srcp.at[wid, pl.ds(ph * QB2, QB2)], srcv)
        pltpu.sync_copy(dstp.at[wid, pl.ds(ph * QB2, QB2)], dstv)
        for b, (rows, sem) in enumerate(ring):
            pltpu.async_copy(g_hbm.at[srcv.at[b]], rows, sem)

        def step(jj, _):
            for b, (rows, sem) in enumerate(ring):
                batch = jj * 4 + b
                pltpu.make_async_copy(g_hbm.at[srcv.at[batch]], rows, sem).wait()
                pltpu.sync_copy(rows, acc_sh.at[dstv.at[batch]], add=True)

                @pl.when(batch + 4 < QB2)
                def _():
                    pltpu.async_copy(g_hbm.at[srcv.at[batch + 4]], rows, sem)

            return 0

        lax.fori_loop(0, QB2 // 4, step, 0)
    plsc.subcore_barrier()

    sl = pl.ds(sid * RPT, RPT)

    @pl.when(cid == 0)
    def _():
        pltpu.sync_copy(acc_sh.at[sl], out0.at[sl])

    @pl.when(cid == 1)
    def _():
        pltpu.sync_copy(acc_sh.at[sl], out1.at[sl])


def _edge_pass(g, srcp, dstp):
    mesh = plsc.VectorSubcoreMesh(
        core_axis_name="c", subcore_axis_name="s", num_cores=NC, num_subcores=NS
    )
    return pl.kernel(
        _edge_body,
        out_type=(
            jax.ShapeDtypeStruct((NPAD, D), f32),
            jax.ShapeDtypeStruct((NPAD, D), f32),
        ),
        mesh=mesh,
        scratch_types=[
            pltpu.VMEM((QB2, EB2), i32),
            pltpu.VMEM((QB2, EB2), i32),
            pltpu.VMEM((EB2, D), f32),
            pltpu.VMEM((EB2, D), f32),
            pltpu.VMEM((EB2, D), f32),
            pltpu.VMEM((EB2, D), f32),
            pltpu.VMEM_SHARED((NPAD, D), f32),
            pltpu.SemaphoreType.DMA,
            pltpu.SemaphoreType.DMA,
            pltpu.SemaphoreType.DMA,
            pltpu.SemaphoreType.DMA,
        ],
    )(g, srcp, dstp)


_BR = 2000
_GRID = N // _BR


def _b_body(x_ref, wc_ref, c0_ref, c1_ref, g_ref, d_ref):
    h = jnp.dot(x_ref[...], wc_ref[...], preferred_element_type=f32)
    deg = 1.0 + c0_ref[...] + c1_ref[...]
    dv = lax.rsqrt(deg)
    d_ref[...] = dv
    g_ref[...] = h * dv


def _premix(state, Wc, c0, c1):
    return pl.pallas_call(
        _b_body,
        grid=(_GRID,),
        in_specs=[
            pl.BlockSpec((_BR, D), lambda i: (i, 0)),
            pl.BlockSpec((D, D), lambda i: (0, 0)),
            pl.BlockSpec((_BR, 1), lambda i: (i, 0)),
            pl.BlockSpec((_BR, 1), lambda i: (i, 0)),
        ],
        out_specs=[
            pl.BlockSpec((_BR, D), lambda i: (i, 0)),
            pl.BlockSpec((_BR, 1), lambda i: (i, 0)),
        ],
        out_shape=[
            jax.ShapeDtypeStruct((N, D), f32),
            jax.ShapeDtypeStruct((N, 1), f32),
        ],
    )(state, Wc, c0, c1)


def _d_body(a0, a1, g, x0, d, bc, w1, b1, w2, b2, w3, b3, conc_ref, sum_ref, acc):
    i = pl.program_id(0)
    a = a0[...] + a1[...] + g[...]
    out = jnp.maximum(d[...] * a + bc[...], 0.0)
    x = out + x0[...]
    y = jnp.dot(x, w1[...], preferred_element_type=f32) + b1[...]
    y = jnp.where(y >= 0, y, 0.01 * y)
    y = jnp.dot(y, w2[...], preferred_element_type=f32) + b2[...]
    y = jnp.where(y >= 0, y, 0.01 * y)
    z = jnp.dot(y, w3[...], preferred_element_type=f32) + b3[...]
    sp = jnp.maximum(z, 0.0) + jnp.log(1.0 + jnp.exp(-jnp.abs(z)))
    conc_ref[...] = sp
    prev = jnp.where(i == 0, 0.0, acc[0, 0])
    tot = prev + jnp.sum(sp)
    acc[0, 0] = tot
    sum_ref[...] = jnp.broadcast_to(tot, (1, 1))


def _head(a0, a1, g, state, d, bc2, W1, b12, W2, b22, W3, b32):
    return pl.pallas_call(
        _d_body,
        grid=(_GRID,),
        in_specs=[
            pl.BlockSpec((_BR, D), lambda i: (i, 0)),
            pl.BlockSpec((_BR, D), lambda i: (i, 0)),
            pl.BlockSpec((_BR, D), lambda i: (i, 0)),
            pl.BlockSpec((_BR, D), lambda i: (i, 0)),
            pl.BlockSpec((_BR, 1), lambda i: (i, 0)),
            pl.BlockSpec((1, D), lambda i: (0, 0)),
            pl.BlockSpec((D, H), lambda i: (0, 0)),
            pl.BlockSpec((1, H), lambda i: (0, 0)),
            pl.BlockSpec((H, H), lambda i: (0, 0)),
            pl.BlockSpec((1, H), lambda i: (0, 0)),
            pl.BlockSpec((H, 1), lambda i: (0, 0)),
            pl.BlockSpec((1, 1), lambda i: (0, 0)),
        ],
        out_specs=[
            pl.BlockSpec((_BR, 1), lambda i: (i, 0)),
            pl.BlockSpec((1, 1), lambda i: (0, 0)),
        ],
        out_shape=[
            jax.ShapeDtypeStruct((N, 1), f32),
            jax.ShapeDtypeStruct((1, 1), f32),
        ],
        scratch_shapes=[pltpu.SMEM((1, 1), f32)],
    )(a0, a1, g, state, d, bc2, W1, b12, W2, b22, W3, b32)


def _e_body(conc_ref, s_ref, out_ref):
    out_ref[...] = conc_ref[...] / (s_ref[...] + 1e-20)


def _normalize(conc, ssum):
    return pl.pallas_call(
        _e_body,
        grid=(_GRID,),
        in_specs=[
            pl.BlockSpec((_BR, 1), lambda i: (i, 0)),
            pl.BlockSpec((1, 1), lambda i: (0, 0)),
        ],
        out_specs=pl.BlockSpec((_BR, 1), lambda i: (i, 0)),
        out_shape=jax.ShapeDtypeStruct((N, 1), f32),
    )(conc, ssum)


def kernel(state, edge_index, deterministic, Wc, bc, W1, b1, W2, b2, W3, b3):
    src = edge_index[0].reshape(NW, EPT)
    dst = edge_index[1].reshape(NW, EPT)
    padw = NB * EB - EPT
    srcf = jnp.concatenate([src, jnp.zeros((NW, padw), i32)], axis=1)
    dstf = jnp.concatenate([dst, jnp.full((NW, padw), N, i32)], axis=1)
    srcp = srcf.reshape(NW, NB2, EB2)
    dstp = dstf.reshape(NW, NB2, EB2)

    c0, c1 = _count_deg(dstf.reshape(NW, NB, EB))
    g, d = _premix(state, Wc, c0.reshape(NPAD, 1), c1.reshape(NPAD, 1))
    a0, a1 = _edge_pass(g, srcp, dstp)
    conc, ssum = _head(
        a0, a1, g, state, d,
        bc.reshape(1, D),
        W1, b1.reshape(1, H),
        W2, b2.reshape(1, H),
        W3, b3.reshape(1, 1),
    )
    act = _normalize(conc, ssum)
    return act[:, 0].reshape(N // ACT, ACT)


# speedup vs baseline: 14.6027x; 1.0085x over previous
import jax
import jax.numpy as jnp
from jax import lax
from jax.experimental import pallas as pl
from jax.experimental.pallas import tpu as pltpu
from jax.experimental.pallas import tpu_sc as plsc

N = 10000
E = 320000
D = 128
H = 32
ACT = 8

NC = 2
NS = 16
NW = NC * NS
NPAD = 10240
RPT = NPAD // NS
EPT = E // NW
EB = 128
NB = 80
EB2 = 64
NB2 = 160
QB2 = NB2 // 4

f32 = jnp.float32
i32 = jnp.int32


def _deg_body(dstp, cnt0, cnt1, idx_v, ones_v, zb_v, cnt_sh):
    cid = lax.axis_index("c")
    sid = lax.axis_index("s")
    wid = cid * NS + sid

    def zstep(j, _):
        zb_v[pl.ds(j * 16, 16)] = jnp.zeros((16,), f32)
        return 0

    lax.fori_loop(0, RPT // 16, zstep, 0)
    pltpu.sync_copy(zb_v, cnt_sh.at[pl.ds(sid * RPT, RPT)])

    def ostep(j, _):
        ones_v[pl.ds(j * 16, 16)] = jnp.ones((16,), f32)
        return 0

    lax.fori_loop(0, EB // 16, ostep, 0)
    pltpu.sync_copy(dstp.at[wid], idx_v)
    plsc.subcore_barrier()

    def step(j, _):
        pltpu.sync_copy(ones_v, cnt_sh.at[idx_v.at[j]], add=True)
        return 0

    lax.fori_loop(0, NB, step, 0)
    plsc.subcore_barrier()

    sl = pl.ds(sid * RPT, RPT)

    @pl.when(cid == 0)
    def _():
        pltpu.sync_copy(cnt_sh.at[sl], cnt0.at[sl])

    @pl.when(cid == 1)
    def _():
        pltpu.sync_copy(cnt_sh.at[sl], cnt1.at[sl])


def _count_deg(dstp):
    mesh = plsc.VectorSubcoreMesh(
        core_axis_name="c", subcore_axis_name="s", num_cores=NC, num_subcores=NS
    )
    return pl.kernel(
        _deg_body,
        out_type=(
            jax.ShapeDtypeStruct((NPAD,), f32),
            jax.ShapeDtypeStruct((NPAD,), f32),
        ),
        mesh=mesh,
        scratch_types=[
            pltpu.VMEM((NB, EB), i32),
            pltpu.VMEM((EB,), f32),
            pltpu.VMEM((RPT,), f32),
            pltpu.VMEM_SHARED((NPAD,), f32),
        ],
    )(dstp)


def _edge_body(
    g_hbm, srcp, dstp, out0, out1,
    srcv, dstv, rows0, rows1, rows2, rows3, acc_sh, sem0, sem1, sem2, sem3
):
    cid = lax.axis_index("c")
    sid = lax.axis_index("s")
    wid = cid * NS + sid
    ring = ((rows0, sem0), (rows1, sem1), (rows2, sem2), (rows3, sem3))

    def zrow(r, _):
        for k in range(D // 16):
            rows0[r, pl.ds(k * 16, 16)] = jnp.zeros((16,), f32)
            rows1[r, pl.ds(k * 16, 16)] = jnp.zeros((16,), f32)
        return 0

    lax.fori_loop(0, EB2, zrow, 0)
    for k in range(RPT // (2 * EB2)):
        pltpu.sync_copy(rows0, acc_sh.at[pl.ds(sid * RPT + 2 * k * EB2, EB2)])
        pltpu.sync_copy(
            rows1, acc_sh.at[pl.ds(sid * RPT + (2 * k + 1) * EB2, EB2)]
        )

    plsc.subcore_barrier()

    for ph in range(4):
        pltpu.sync_copy(srcp.at[wid, pl.ds(ph * QB2, QB2)], srcv)
        pltpu.sync_copy(dstp.at[wid, pl.ds(ph * QB2, QB2)], dstv)
        for b, (rows, sem) in enumerate(ring):
            pltpu.async_copy(g_hbm.at[srcv.at[b]], rows, sem)

        def step(jj, _):
            for b, (rows, sem) in enumerate(ring):
                batch = jj * 4 + b
                pltpu.make_async_copy(g_hbm.at[srcv.at[batch]], rows, sem).wait()
                pltpu.sync_copy(rows, acc_sh.at[dstv.at[batch]], add=True)

                @pl.when(batch + 4 < QB2)
                def _():
                    pltpu.async_copy(g_hbm.at[srcv.at[batch + 4]], rows, sem)

            return 0

        lax.fori_loop(0, QB2 // 4, step, 0)
    plsc.subcore_barrier()

    sl = pl.ds(sid * RPT, RPT)

    @pl.when(cid == 0)
    def _():
        pltpu.sync_copy(acc_sh.at[sl], out0.at[sl])

    @pl.when(cid == 1)
    def _():
        pltpu.sync_copy(acc_sh.at[sl], out1.at[sl])


def _edge_pass(g, srcp, dstp):
    mesh = plsc.VectorSubcoreMesh(
        core_axis_name="c", subcore_axis_name="s", num_cores=NC, num_subcores=NS
    )
    return pl.kernel(
        _edge_body,
        out_type=(
            jax.ShapeDtypeStruct((NPAD, D), f32),
            jax.ShapeDtypeStruct((NPAD, D), f32),
        ),
        mesh=mesh,
        scratch_types=[
            pltpu.VMEM((QB2, EB2), i32),
            pltpu.VMEM((QB2, EB2), i32),
            pltpu.VMEM((EB2, D), f32),
            pltpu.VMEM((EB2, D), f32),
            pltpu.VMEM((EB2, D), f32),
            pltpu.VMEM((EB2, D), f32),
            pltpu.VMEM_SHARED((NPAD, D), f32),
            pltpu.SemaphoreType.DMA,
            pltpu.SemaphoreType.DMA,
            pltpu.SemaphoreType.DMA,
            pltpu.SemaphoreType.DMA,
        ],
    )(g, srcp, dstp)


_BR = 2000
_GRID = N // _BR


def _b_body(x_ref, wc_ref, c0_ref, c1_ref, g_ref, d_ref):
    h = jnp.dot(x_ref[...], wc_ref[...], preferred_element_type=f32)
    deg = 1.0 + c0_ref[...] + c1_ref[...]
    dv = lax.rsqrt(deg)
    d_ref[...] = dv
    g_ref[...] = h * dv


def _premix(state, Wc, c0, c1):
    return pl.pallas_call(
        _b_body,
        grid=(_GRID,),
        in_specs=[
            pl.BlockSpec((_BR, D), lambda i: (i, 0)),
            pl.BlockSpec((D, D), lambda i: (0, 0)),
            pl.BlockSpec((_BR, 1), lambda i: (i, 0)),
            pl.BlockSpec((_BR, 1), lambda i: (i, 0)),
        ],
        out_specs=[
            pl.BlockSpec((_BR, D), lambda i: (i, 0)),
            pl.BlockSpec((_BR, 1), lambda i: (i, 0)),
        ],
        out_shape=[
            jax.ShapeDtypeStruct((N, D), f32),
            jax.ShapeDtypeStruct((N, 1), f32),
        ],
    )(state, Wc, c0, c1)


def _d_body(a0, a1, g, x0, d, bc, w1, b1, w2, b2, w3, b3, act_ref, conc_vmem, acc):
    p = pl.program_id(0)
    i = pl.program_id(1)

    @pl.when(p == 0)
    def _():
        a = a0[...] + a1[...] + g[...]
        out = jnp.maximum(d[...] * a + bc[...], 0.0)
        x = out + x0[...]
        y = jnp.dot(x, w1[...], preferred_element_type=f32) + b1[...]
        y = jnp.where(y >= 0, y, 0.01 * y)
        y = jnp.dot(y, w2[...], preferred_element_type=f32) + b2[...]
        y = jnp.where(y >= 0, y, 0.01 * y)
        z = jnp.dot(y, w3[...], preferred_element_type=f32) + b3[...]
        sp = jnp.maximum(z, 0.0) + jnp.log(1.0 + jnp.exp(-jnp.abs(z)))
        conc_vmem[pl.ds(i * _BR, _BR), :] = sp
        prev = jnp.where(i == 0, 0.0, acc[0, 0])
        acc[0, 0] = prev + jnp.sum(sp)

    @pl.when(p == 1)
    def _():
        act_ref[...] = conc_vmem[pl.ds(i * _BR, _BR), :] / (acc[0, 0] + 1e-20)


def _head(a0, a1, g, state, d, bc2, W1, b12, W2, b22, W3, b32):
    blk = lambda p, i: (i * (1 - p) + (_GRID - 1) * p, 0)
    fix = lambda p, i: (0, 0)
    return pl.pallas_call(
        _d_body,
        grid=(2, _GRID),
        in_specs=[
            pl.BlockSpec((_BR, D), blk),
            pl.BlockSpec((_BR, D), blk),
            pl.BlockSpec((_BR, D), blk),
            pl.BlockSpec((_BR, D), blk),
            pl.BlockSpec((_BR, 1), blk),
            pl.BlockSpec((1, D), fix),
            pl.BlockSpec((D, H), fix),
            pl.BlockSpec((1, H), fix),
            pl.BlockSpec((H, H), fix),
            pl.BlockSpec((1, H), fix),
            pl.BlockSpec((H, 1), fix),
            pl.BlockSpec((1, 1), fix),
        ],
        out_specs=pl.BlockSpec((_BR, 1), lambda p, i: (i, 0)),
        out_shape=jax.ShapeDtypeStruct((N, 1), f32),
        scratch_shapes=[pltpu.VMEM((N, 1), f32), pltpu.SMEM((1, 1), f32)],
    )(a0, a1, g, state, d, bc2, W1, b12, W2, b22, W3, b32)


def kernel(state, edge_index, deterministic, Wc, bc, W1, b1, W2, b2, W3, b3):
    src = edge_index[0].reshape(NW, EPT)
    dst = edge_index[1].reshape(NW, EPT)
    padw = NB * EB - EPT
    srcf = jnp.concatenate([src, jnp.zeros((NW, padw), i32)], axis=1)
    dstf = jnp.concatenate([dst, jnp.full((NW, padw), N, i32)], axis=1)
    srcp = srcf.reshape(NW, NB2, EB2)
    dstp = dstf.reshape(NW, NB2, EB2)

    c0, c1 = _count_deg(dstf.reshape(NW, NB, EB))
    g, d = _premix(state, Wc, c0.reshape(NPAD, 1), c1.reshape(NPAD, 1))
    a0, a1 = _edge_pass(g, srcp, dstp)
    act = _head(
        a0, a1, g, state, d,
        bc.reshape(1, D),
        W1, b1.reshape(1, H),
        W2, b2.reshape(1, H),
        W3, b3.reshape(1, 1),
    )
    return act[:, 0].reshape(N // ACT, ACT)
